# baseline (device time: 34763 ns/iter reference)
import jax
import jax.numpy as jnp
from jax import lax
from jax.experimental import pallas as pl
from jax.experimental.pallas import tpu as pltpu

BLOCK_M = 512


def kernel(x):
    m, n = x.shape
    grid = m // BLOCK_M

    def body(x_ref, out_ref, send_buf, recv_buf, send_sem, recv_sem):
        i = pl.program_id(0)
        my_x = lax.axis_index("x")
        my_y = lax.axis_index("y")
        peer = (my_x, 1 - my_y)

        barrier_sem = pltpu.get_barrier_semaphore()

        @pl.when(i == 0)
        def _():
            pl.semaphore_signal(
                barrier_sem, inc=1,
                device_id=peer, device_id_type=pl.DeviceIdType.MESH,
            )

        send_buf[pl.ds(i * BLOCK_M, BLOCK_M), :] = jnp.max(
            x_ref[:, :], axis=1, keepdims=True
        )

        @pl.when(i == grid - 1)
        def _():
            pl.semaphore_wait(barrier_sem, 1)
            rdma = pltpu.make_async_remote_copy(
                src_ref=send_buf,
                dst_ref=recv_buf,
                send_sem=send_sem,
                recv_sem=recv_sem,
                device_id=peer,
                device_id_type=pl.DeviceIdType.MESH,
            )
            rdma.start()
            rdma.wait()
            out_ref[:, :] = jnp.maximum(send_buf[:, :], recv_buf[:, :])

    return pl.pallas_call(
        body,
        grid=(grid,),
        out_shape=jax.ShapeDtypeStruct((m, 1), jnp.float32),
        in_specs=[
            pl.BlockSpec((BLOCK_M, n), lambda i: (i, 0), memory_space=pltpu.VMEM)
        ],
        out_specs=pl.BlockSpec((m, 1), lambda i: (0, 0), memory_space=pltpu.VMEM),
        scratch_shapes=[
            pltpu.VMEM((m, 1), jnp.float32),
            pltpu.VMEM((m, 1), jnp.float32),
            pltpu.SemaphoreType.DMA,
            pltpu.SemaphoreType.DMA,
        ],
        compiler_params=pltpu.CompilerParams(collective_id=0),
    )(x)


# device time: 11648 ns/iter; 2.9845x vs baseline; 2.9845x over previous
import jax
import jax.numpy as jnp
from jax import lax
from jax.experimental import pallas as pl
from jax.experimental.pallas import tpu as pltpu

BLOCK_M = 512
LANES = 128


def kernel(x):
    m, n = x.shape
    grid = m // BLOCK_M

    def body(x_ref, out_ref, send_buf, recv_buf, send_sem, recv_sem):
        i = pl.program_id(0)
        my_x = lax.axis_index("x")
        my_y = lax.axis_index("y")
        peer = (my_x, 1 - my_y)

        barrier_sem = pltpu.get_barrier_semaphore()

        @pl.when(i == 0)
        def _():
            pl.semaphore_signal(
                barrier_sem, inc=1,
                device_id=peer, device_id_type=pl.DeviceIdType.MESH,
            )

        w = x_ref[:, 0:LANES]
        for k in range(1, n // LANES):
            w = jnp.maximum(w, x_ref[:, k * LANES:(k + 1) * LANES])
        wt = jnp.max(w.T, axis=0, keepdims=True)
        send_buf[pl.ds(i, 1), :] = wt

        @pl.when(i == grid - 1)
        def _():
            pl.semaphore_wait(barrier_sem, 1)
            rdma = pltpu.make_async_remote_copy(
                src_ref=send_buf,
                dst_ref=recv_buf,
                send_sem=send_sem,
                recv_sem=recv_sem,
                device_id=peer,
                device_id_type=pl.DeviceIdType.MESH,
            )
            rdma.start()
            rdma.wait()
            out_ref[:, :] = jnp.maximum(send_buf[:, :], recv_buf[:, :])

    packed = pl.pallas_call(
        body,
        grid=(grid,),
        out_shape=jax.ShapeDtypeStruct((grid, BLOCK_M), jnp.float32),
        in_specs=[
            pl.BlockSpec((BLOCK_M, n), lambda i: (i, 0), memory_space=pltpu.VMEM)
        ],
        out_specs=pl.BlockSpec(
            (grid, BLOCK_M), lambda i: (0, 0), memory_space=pltpu.VMEM
        ),
        scratch_shapes=[
            pltpu.VMEM((grid, BLOCK_M), jnp.float32),
            pltpu.VMEM((grid, BLOCK_M), jnp.float32),
            pltpu.SemaphoreType.DMA,
            pltpu.SemaphoreType.DMA,
        ],
        compiler_params=pltpu.CompilerParams(collective_id=0),
    )(x)
    return packed.reshape(m, 1)


# device time: 10932 ns/iter; 3.1799x vs baseline; 1.0655x over previous
import jax
import jax.numpy as jnp
from jax import lax
from jax.experimental import pallas as pl
from jax.experimental.pallas import tpu as pltpu

LANES = 128
PACK_COLS = 512


def kernel(x):
    m, n = x.shape
    pack_rows = m // PACK_COLS

    def body(x_ref, out_ref, send_buf, recv_buf, send_sem, recv_sem):
        my_x = lax.axis_index("x")
        my_y = lax.axis_index("y")
        peer = (my_x, 1 - my_y)

        barrier_sem = pltpu.get_barrier_semaphore()
        pl.semaphore_signal(
            barrier_sem, inc=1,
            device_id=peer, device_id_type=pl.DeviceIdType.MESH,
        )

        w = x_ref[:, 0:LANES]
        for k in range(1, n // LANES):
            w = jnp.maximum(w, x_ref[:, k * LANES:(k + 1) * LANES])
        for j in range(pack_rows):
            wj = w[j * PACK_COLS:(j + 1) * PACK_COLS, :]
            send_buf[pl.ds(j, 1), :] = jnp.max(wj.T, axis=0, keepdims=True)

        pl.semaphore_wait(barrier_sem, 1)
        rdma = pltpu.make_async_remote_copy(
            src_ref=send_buf,
            dst_ref=recv_buf,
            send_sem=send_sem,
            recv_sem=recv_sem,
            device_id=peer,
            device_id_type=pl.DeviceIdType.MESH,
        )
        rdma.start()
        rdma.wait()
        out_ref[:, :] = jnp.maximum(send_buf[:, :], recv_buf[:, :])

    packed = pl.pallas_call(
        body,
        out_shape=jax.ShapeDtypeStruct((pack_rows, PACK_COLS), jnp.float32),
        in_specs=[pl.BlockSpec(memory_space=pltpu.VMEM)],
        out_specs=pl.BlockSpec(memory_space=pltpu.VMEM),
        scratch_shapes=[
            pltpu.VMEM((pack_rows, PACK_COLS), jnp.float32),
            pltpu.VMEM((pack_rows, PACK_COLS), jnp.float32),
            pltpu.SemaphoreType.DMA,
            pltpu.SemaphoreType.DMA,
        ],
        compiler_params=pltpu.CompilerParams(collective_id=0),
    )(x)
    return packed.reshape(m, 1)
